# Initial kernel scaffold; baseline (speedup 1.0000x reference)
import jax
import jax.numpy as jnp
from jax.experimental import pallas as pl


def kernel(ft_tf, ft_gene, adj_tf_tf, adj_tf_gene, adj_gene_tf, adj_gene_gene, params):
    raise NotImplementedError("write your pallas kernel here")


# speedup vs baseline: 1.4018x; 1.4018x over previous
import jax
import jax.numpy as jnp
from jax.experimental import pallas as pl

F32 = jnp.float32
TILE = 384
D1 = 128
D2 = 64


def _stage1_body(a1_ref, a2_ref, xtf_ref, xgene_ref, xs_ref, ew_ref, ln_ref,
                 wc_ref, ztf_ref, zgene_ref, selfc_ref):
    t = a1_ref.shape[0]
    b = xs_ref.shape[1]
    ytf = jnp.dot(a1_ref[...], xtf_ref[...], preferred_element_type=F32)
    ygene = jnp.dot(a2_ref[...], xgene_ref[...], preferred_element_type=F32)
    ones = jnp.ones((t, 1), F32)
    c = jnp.concatenate([xs_ref[...], ytf, ygene, ones], axis=1)
    h = jnp.dot(c, ew_ref[...], preferred_element_type=F32)
    g = ln_ref[0:1, :]
    be = ln_ref[1:2, :]
    wc = wc_ref[...]
    for j in range(b):
        hj = h[:, j * D1:(j + 1) * D1]
        mu = jnp.mean(hj, axis=1, keepdims=True)
        dd = hj - mu
        var = jnp.mean(dd * dd, axis=1, keepdims=True)
        xn = dd * jax.lax.rsqrt(var + 1e-5) * g + be
        xn = jnp.where(xn > 0, xn, jnp.exp(xn) - 1.0)
        z = jnp.dot(xn, wc, preferred_element_type=F32)
        ztf_ref[:, j * D2:(j + 1) * D2] = z[:, 0:D2]
        zgene_ref[:, j * D2:(j + 1) * D2] = z[:, D2:2 * D2]
        selfc_ref[:, j * D2:(j + 1) * D2] = z[:, 2 * D2:3 * D2]


def _stage2_body(a1_ref, a2_ref, ztf_ref, zgene_ref, selfc_ref, w2_ref, p_ref):
    t = a1_ref.shape[0]
    b = p_ref.shape[0]
    msg = jnp.dot(a1_ref[...], ztf_ref[...], preferred_element_type=F32)
    msg = msg + jnp.dot(a2_ref[...], zgene_ref[...], preferred_element_type=F32)
    sc = selfc_ref[...]
    w = w2_ref[...]
    bias = w[0:1, :]
    g = w[1:2, :]
    be = w[2:3, :]
    for j in range(b):
        hj = (msg[:, j * D2:(j + 1) * D2] + sc[:, j * D2:(j + 1) * D2]) * (1.0 / 3.0) + bias
        mu = jnp.mean(hj, axis=1, keepdims=True)
        dd = hj - mu
        var = jnp.mean(dd * dd, axis=1, keepdims=True)
        xn = dd * jax.lax.rsqrt(var + 1e-5) * g + be
        xn = jnp.where(xn > 0, xn, jnp.exp(xn) - 1.0)
        p_ref[j] = jnp.max(xn.reshape(t // 8, 8, D2), axis=1)


def _head_body(flat_ref, xc_ref, gew_ref, geb_ref, recw_ref, recb_ref,
               f1w_ref, f1b_ref, f2w_ref, f2b_ref, cw_ref, cb_ref,
               logits_ref, dec_ref, cell_ref):
    xh = jnp.dot(flat_ref[...], gew_ref[...], preferred_element_type=F32)
    xh = jnp.maximum(xh + geb_ref[...], 0.0)
    dec_ref[...] = jnp.dot(xh, recw_ref[...], preferred_element_type=F32) + recb_ref[...]
    xnn = jnp.dot(xc_ref[...], f1w_ref[...], preferred_element_type=F32)
    xnn = jnp.maximum(xnn + f1b_ref[...], 0.0)
    xnn = jnp.dot(xnn, f2w_ref[...], preferred_element_type=F32)
    xnn = jnp.maximum(xnn + f2b_ref[...], 0.0)
    cell = jnp.concatenate([xh, xnn], axis=1)
    cell_ref[...] = cell
    logits_ref[...] = jnp.dot(cell, cw_ref[...], preferred_element_type=F32) + cb_ref[...]


def _stage1(a1, a2, xs, xtf, xgene, ew, ln, wc):
    s = a1.shape[0]
    b = xs.shape[1]
    n = s // TILE
    zspec = pl.BlockSpec((TILE, b * D2), lambda i: (i, 0))
    return pl.pallas_call(
        _stage1_body,
        grid=(n,),
        in_specs=[
            pl.BlockSpec((TILE, a1.shape[1]), lambda i: (i, 0)),
            pl.BlockSpec((TILE, a2.shape[1]), lambda i: (i, 0)),
            pl.BlockSpec(xtf.shape, lambda i: (0, 0)),
            pl.BlockSpec(xgene.shape, lambda i: (0, 0)),
            pl.BlockSpec((TILE, b), lambda i: (i, 0)),
            pl.BlockSpec(ew.shape, lambda i: (0, 0)),
            pl.BlockSpec(ln.shape, lambda i: (0, 0)),
            pl.BlockSpec(wc.shape, lambda i: (0, 0)),
        ],
        out_specs=[zspec, zspec, zspec],
        out_shape=[jax.ShapeDtypeStruct((s, b * D2), F32)] * 3,
    )(a1, a2, xtf, xgene, xs, ew, ln, wc)


def _stage2(a1, a2, ztf, zgene, selfc, w2, b):
    s = a1.shape[0]
    n = s // TILE
    return pl.pallas_call(
        _stage2_body,
        grid=(n,),
        in_specs=[
            pl.BlockSpec((TILE, a1.shape[1]), lambda i: (i, 0)),
            pl.BlockSpec((TILE, a2.shape[1]), lambda i: (i, 0)),
            pl.BlockSpec(ztf.shape, lambda i: (0, 0)),
            pl.BlockSpec(zgene.shape, lambda i: (0, 0)),
            pl.BlockSpec((TILE, b * D2), lambda i: (i, 0)),
            pl.BlockSpec(w2.shape, lambda i: (0, 0)),
        ],
        out_specs=pl.BlockSpec((b, TILE // 8, D2), lambda i: (0, i, 0)),
        out_shape=jax.ShapeDtypeStruct((b, s // 8, D2), F32),
    )(a1, a2, ztf, zgene, selfc, w2)


def kernel(ft_tf, ft_gene, adj_tf_tf, adj_tf_gene, adj_gene_tf, adj_gene_gene, params):
    b = ft_tf.shape[0]
    xtf = ft_tf[:, :, 0].T
    xgene = ft_gene[:, :, 0].T
    xc = jnp.concatenate([ft_tf[:, :, 0], ft_gene[:, :, 0]], axis=1)

    p1 = params['hgc1']
    p2 = params['hgc2']

    def pack_ew(k):
        w = jnp.stack([p1[k]['w_self'][0], p1[k]['w_rel_tf'][0],
                       p1[k]['w_rel_gene'][0]], axis=0) / 3.0
        eye = jnp.eye(b, dtype=F32)
        e = jnp.einsum('jk,rd->rjkd', eye, w).reshape(3 * b, b * D1)
        bias_t = jnp.tile(p1[k]['bias'][0], b)[None]
        return jnp.concatenate([e, bias_t], axis=0)

    ln1 = jnp.concatenate([params['ln1_g'][None], params['ln1_b'][None],
                           jnp.zeros((6, D1), F32)], axis=0)

    def pack_wc(src):
        return jnp.concatenate(
            [p2['tf']['w_rel_' + src], p2['gene']['w_rel_' + src], p2[src]['w_self']],
            axis=1)

    def packb(k):
        return jnp.concatenate(
            [p2[k]['bias'], params['ln2_g'][None], params['ln2_b'][None],
             jnp.zeros((5, D2), F32)], axis=0)

    ztf_a, zgene_a, self_tf = _stage1(adj_tf_tf, adj_tf_gene, xtf, xtf, xgene,
                                      pack_ew('tf'), ln1, pack_wc('tf'))
    ztf_b, zgene_b, self_gene = _stage1(adj_gene_tf, adj_gene_gene, xgene, xtf, xgene,
                                        pack_ew('gene'), ln1, pack_wc('gene'))

    p_tf = _stage2(adj_tf_tf, adj_tf_gene, ztf_a, ztf_b, self_tf, packb('tf'), b)
    p_gene = _stage2(adj_gene_tf, adj_gene_gene, zgene_a, zgene_b, self_gene,
                     packb('gene'), b)

    flat = jnp.concatenate([p_tf, p_gene], axis=1).reshape(b, -1)

    logits, dec, cell = pl.pallas_call(
        _head_body,
        out_shape=[
            jax.ShapeDtypeStruct((b, params['cls_b'].shape[0]), F32),
            jax.ShapeDtypeStruct((b, params['rec_b'].shape[0]), F32),
            jax.ShapeDtypeStruct((b, 2 * params['fc2_b'].shape[0]), F32),
        ],
    )(flat, xc, params['ge_W'], params['ge_b'][None], params['rec_W'],
      params['rec_b'][None], params['fc1_W'], params['fc1_b'][None],
      params['fc2_W'], params['fc2_b'][None], params['cls_W'], params['cls_b'][None])
    return (logits, dec, cell)


# speedup vs baseline: 1.4636x; 1.0441x over previous
import jax
import jax.numpy as jnp
from jax.experimental import pallas as pl

F32 = jnp.float32
TILE = 384
D1 = 128
D2 = 64
BF16 = jnp.bfloat16


def _stage1_body(a1_ref, a2_ref, xtf_ref, xgene_ref, xs_ref, ew_ref, ln_ref,
                 wc_ref, ztf_ref, zgene_ref, selfc_ref):
    t = a1_ref.shape[0]
    b = xs_ref.shape[1]
    ytf = jnp.dot(a1_ref[...].astype(BF16), xtf_ref[...].astype(BF16),
                  preferred_element_type=F32)
    ygene = jnp.dot(a2_ref[...].astype(BF16), xgene_ref[...].astype(BF16),
                    preferred_element_type=F32)
    ones = jnp.ones((t, 1), F32)
    c = jnp.concatenate([xs_ref[...], ytf, ygene, ones], axis=1)
    h = jnp.dot(c.astype(BF16), ew_ref[...].astype(BF16),
                preferred_element_type=F32)
    g = ln_ref[0:1, :]
    be = ln_ref[1:2, :]
    wc = wc_ref[...].astype(BF16)
    for j in range(b):
        hj = h[:, j * D1:(j + 1) * D1]
        mu = jnp.mean(hj, axis=1, keepdims=True)
        dd = hj - mu
        var = jnp.mean(dd * dd, axis=1, keepdims=True)
        xn = dd * jax.lax.rsqrt(var + 1e-5) * g + be
        xn = jnp.where(xn > 0, xn, jnp.exp(xn) - 1.0)
        z = jnp.dot(xn.astype(BF16), wc, preferred_element_type=F32)
        ztf_ref[:, j * D2:(j + 1) * D2] = z[:, 0:D2].astype(BF16)
        zgene_ref[:, j * D2:(j + 1) * D2] = z[:, D2:2 * D2].astype(BF16)
        selfc_ref[:, j * D2:(j + 1) * D2] = z[:, 2 * D2:3 * D2]


def _stage2_body(a1_ref, a2_ref, ztf_ref, zgene_ref, selfc_ref, w2_ref, p_ref):
    t = a1_ref.shape[0]
    b = p_ref.shape[0]
    msg = jnp.dot(a1_ref[...].astype(BF16), ztf_ref[...], preferred_element_type=F32)
    msg = msg + jnp.dot(a2_ref[...].astype(BF16), zgene_ref[...],
                        preferred_element_type=F32)
    sc = selfc_ref[...]
    w = w2_ref[...]
    bias = w[0:1, :]
    g = w[1:2, :]
    be = w[2:3, :]
    for j in range(b):
        hj = (msg[:, j * D2:(j + 1) * D2] + sc[:, j * D2:(j + 1) * D2]) * (1.0 / 3.0) + bias
        mu = jnp.mean(hj, axis=1, keepdims=True)
        dd = hj - mu
        var = jnp.mean(dd * dd, axis=1, keepdims=True)
        xn = dd * jax.lax.rsqrt(var + 1e-5) * g + be
        xn = jnp.where(xn > 0, xn, jnp.exp(xn) - 1.0)
        p_ref[j] = jnp.max(xn.reshape(t // 8, 8, D2), axis=1)


def _head_body(flat_ref, xc_ref, gew_ref, geb_ref, recw_ref, recb_ref,
               f1w_ref, f1b_ref, f2w_ref, f2b_ref, cw_ref, cb_ref,
               logits_ref, dec_ref, cell_ref):
    xh = jnp.dot(flat_ref[...].astype(BF16), gew_ref[...].astype(BF16),
                 preferred_element_type=F32)
    xh = jnp.maximum(xh + geb_ref[...], 0.0)
    dec_ref[...] = jnp.dot(xh.astype(BF16), recw_ref[...].astype(BF16),
                           preferred_element_type=F32) + recb_ref[...]
    xnn = jnp.dot(xc_ref[...].astype(BF16), f1w_ref[...].astype(BF16),
                  preferred_element_type=F32)
    xnn = jnp.maximum(xnn + f1b_ref[...], 0.0)
    xnn = jnp.dot(xnn.astype(BF16), f2w_ref[...].astype(BF16),
                  preferred_element_type=F32)
    xnn = jnp.maximum(xnn + f2b_ref[...], 0.0)
    cell = jnp.concatenate([xh, xnn], axis=1)
    cell_ref[...] = cell
    logits_ref[...] = jnp.dot(cell.astype(BF16), cw_ref[...].astype(BF16),
                              preferred_element_type=F32) + cb_ref[...]


def _stage1(a1, a2, xs, xtf, xgene, ew, ln, wc):
    s = a1.shape[0]
    b = xs.shape[1]
    n = s // TILE
    zspec = pl.BlockSpec((TILE, b * D2), lambda i: (i, 0))
    return pl.pallas_call(
        _stage1_body,
        grid=(n,),
        in_specs=[
            pl.BlockSpec((TILE, a1.shape[1]), lambda i: (i, 0)),
            pl.BlockSpec((TILE, a2.shape[1]), lambda i: (i, 0)),
            pl.BlockSpec(xtf.shape, lambda i: (0, 0)),
            pl.BlockSpec(xgene.shape, lambda i: (0, 0)),
            pl.BlockSpec((TILE, b), lambda i: (i, 0)),
            pl.BlockSpec(ew.shape, lambda i: (0, 0)),
            pl.BlockSpec(ln.shape, lambda i: (0, 0)),
            pl.BlockSpec(wc.shape, lambda i: (0, 0)),
        ],
        out_specs=[zspec, zspec, zspec],
        out_shape=[jax.ShapeDtypeStruct((s, b * D2), BF16),
                   jax.ShapeDtypeStruct((s, b * D2), BF16),
                   jax.ShapeDtypeStruct((s, b * D2), F32)],
    )(a1, a2, xtf, xgene, xs, ew, ln, wc)


def _stage2(a1, a2, ztf, zgene, selfc, w2, b):
    s = a1.shape[0]
    n = s // TILE
    return pl.pallas_call(
        _stage2_body,
        grid=(n,),
        in_specs=[
            pl.BlockSpec((TILE, a1.shape[1]), lambda i: (i, 0)),
            pl.BlockSpec((TILE, a2.shape[1]), lambda i: (i, 0)),
            pl.BlockSpec(ztf.shape, lambda i: (0, 0)),
            pl.BlockSpec(zgene.shape, lambda i: (0, 0)),
            pl.BlockSpec((TILE, b * D2), lambda i: (i, 0)),
            pl.BlockSpec(w2.shape, lambda i: (0, 0)),
        ],
        out_specs=pl.BlockSpec((b, TILE // 8, D2), lambda i: (0, i, 0)),
        out_shape=jax.ShapeDtypeStruct((b, s // 8, D2), F32),
    )(a1, a2, ztf, zgene, selfc, w2)


def kernel(ft_tf, ft_gene, adj_tf_tf, adj_tf_gene, adj_gene_tf, adj_gene_gene, params):
    b = ft_tf.shape[0]
    xtf = ft_tf[:, :, 0].T
    xgene = ft_gene[:, :, 0].T
    xc = jnp.concatenate([ft_tf[:, :, 0], ft_gene[:, :, 0]], axis=1)

    p1 = params['hgc1']
    p2 = params['hgc2']

    def pack_ew(k):
        w = jnp.stack([p1[k]['w_self'][0], p1[k]['w_rel_tf'][0],
                       p1[k]['w_rel_gene'][0]], axis=0) / 3.0
        eye = jnp.eye(b, dtype=F32)
        e = jnp.einsum('jk,rd->rjkd', eye, w).reshape(3 * b, b * D1)
        bias_t = jnp.tile(p1[k]['bias'][0], b)[None]
        return jnp.concatenate([e, bias_t], axis=0)

    ln1 = jnp.concatenate([params['ln1_g'][None], params['ln1_b'][None],
                           jnp.zeros((6, D1), F32)], axis=0)

    def pack_wc(src):
        return jnp.concatenate(
            [p2['tf']['w_rel_' + src], p2['gene']['w_rel_' + src], p2[src]['w_self']],
            axis=1)

    def packb(k):
        return jnp.concatenate(
            [p2[k]['bias'], params['ln2_g'][None], params['ln2_b'][None],
             jnp.zeros((5, D2), F32)], axis=0)

    ztf_a, zgene_a, self_tf = _stage1(adj_tf_tf, adj_tf_gene, xtf, xtf, xgene,
                                      pack_ew('tf'), ln1, pack_wc('tf'))
    ztf_b, zgene_b, self_gene = _stage1(adj_gene_tf, adj_gene_gene, xgene, xtf, xgene,
                                        pack_ew('gene'), ln1, pack_wc('gene'))

    p_tf = _stage2(adj_tf_tf, adj_tf_gene, ztf_a, ztf_b, self_tf, packb('tf'), b)
    p_gene = _stage2(adj_gene_tf, adj_gene_gene, zgene_a, zgene_b, self_gene,
                     packb('gene'), b)

    flat = jnp.concatenate([p_tf, p_gene], axis=1).reshape(b, -1)

    logits, dec, cell = pl.pallas_call(
        _head_body,
        out_shape=[
            jax.ShapeDtypeStruct((b, params['cls_b'].shape[0]), F32),
            jax.ShapeDtypeStruct((b, params['rec_b'].shape[0]), F32),
            jax.ShapeDtypeStruct((b, 2 * params['fc2_b'].shape[0]), F32),
        ],
    )(flat, xc, params['ge_W'], params['ge_b'][None], params['rec_W'],
      params['rec_b'][None], params['fc1_W'], params['fc1_b'][None],
      params['fc2_W'], params['fc2_b'][None], params['cls_W'], params['cls_b'][None])
    return (logits, dec, cell)


# speedup vs baseline: 2.0851x; 1.4246x over previous
import jax
import jax.numpy as jnp
from jax.experimental import pallas as pl
from jax.experimental.pallas import tpu as pltpu

F32 = jnp.float32
BF16 = jnp.bfloat16
TILE = 384
D1 = 128
D2 = 64


def _norm_elu(h, o, rg, lnb):
    s1 = jnp.dot(h, o, preferred_element_type=F32)
    s2 = jnp.dot(h * h, o, preferred_element_type=F32)
    rs = jax.lax.rsqrt(s2 - s1 * s1 + 1e-5)
    ab = jnp.dot(rs.astype(BF16), rg, preferred_element_type=F32)
    bb = jnp.dot((-s1 * rs).astype(BF16), rg, preferred_element_type=F32)
    xn = h * ab + bb + lnb
    return jnp.where(xn > 0, xn, jnp.exp(xn) - 1.0)


def _stage1_body(a1_ref, a2_ref, xtf_ref, xgene_ref, xs_ref, ew_ref, o_ref,
                 rg_ref, aux_ref, bsc_ref, wblk_ref,
                 ztf_ref, zgene_ref, selfc_ref, a1b_ref, a2b_ref):
    t = a1_ref.shape[0]
    a1b = a1_ref[...].astype(BF16)
    a2b = a2_ref[...].astype(BF16)
    a1b_ref[...] = a1b
    a2b_ref[...] = a2b
    ytf = jnp.dot(a1b, xtf_ref[...].astype(BF16), preferred_element_type=F32)
    ygene = jnp.dot(a2b, xgene_ref[...].astype(BF16), preferred_element_type=F32)
    ones = jnp.ones((t, 1), F32)
    c = jnp.concatenate([xs_ref[...], ytf, ygene, ones], axis=1)
    h = jnp.dot(c.astype(BF16), ew_ref[...].astype(BF16),
                preferred_element_type=F32)
    xn = _norm_elu(h, o_ref[...], rg_ref[...], aux_ref[0:1, :])
    xnb = xn.astype(BF16)
    wp = wblk_ref[...]
    b = xn.shape[1] // D1
    for g in range(b // 2):
        zg = jnp.dot(xnb[:, g * 2 * D1:(g + 1) * 2 * D1], wp,
                     preferred_element_type=F32)
        ztf_ref[:, g * D1:(g + 1) * D1] = zg[:, 0:D1].astype(BF16)
        zgene_ref[:, g * D1:(g + 1) * D1] = zg[:, D1:2 * D1].astype(BF16)
        selfc_ref[:, g * D1:(g + 1) * D1] = (
            zg[:, 2 * D1:3 * D1] + bsc_ref[0:1, g * D1:(g + 1) * D1]).astype(BF16)


def _stage2_body(a1_ref, a2_ref, ztf_ref, zgene_ref, selfc_ref, o_ref, rg_ref,
                 aux_ref, p_ref):
    t = a1_ref.shape[0]
    msg = jnp.dot(a1_ref[...], ztf_ref[...], preferred_element_type=F32)
    msg = msg + jnp.dot(a2_ref[...], zgene_ref[...], preferred_element_type=F32)
    h = msg + selfc_ref[...].astype(F32)
    xn = _norm_elu(h, o_ref[...], rg_ref[...], aux_ref[0:1, :])
    p_ref[...] = jnp.max(xn.reshape(t // 8, 8, xn.shape[1]), axis=1).astype(BF16)


def _head_body(flat_ref, xc_ref, gew_ref, geb_ref, recw_ref, recb_ref,
               f1w_ref, f1b_ref, f2w_ref, f2b_ref, cw_ref, cb_ref,
               logits_ref, dec_ref, cell_ref, acc_ref):
    k = pl.program_id(0)
    nk = pl.num_programs(0)
    part = jnp.dot(flat_ref[...], gew_ref[...].astype(BF16),
                   preferred_element_type=F32)

    @pl.when(k == 0)
    def _():
        acc_ref[...] = part

    @pl.when(k > 0)
    def _():
        acc_ref[...] = acc_ref[...] + part

    @pl.when(k == nk - 1)
    def _():
        xh = jnp.maximum(acc_ref[...] + geb_ref[...], 0.0)
        dec_ref[...] = jnp.dot(xh.astype(BF16), recw_ref[...].astype(BF16),
                               preferred_element_type=F32) + recb_ref[...]
        xnn = jnp.dot(xc_ref[...].astype(BF16), f1w_ref[...].astype(BF16),
                      preferred_element_type=F32)
        xnn = jnp.maximum(xnn + f1b_ref[...], 0.0)
        xnn = jnp.dot(xnn.astype(BF16), f2w_ref[...].astype(BF16),
                      preferred_element_type=F32)
        xnn = jnp.maximum(xnn + f2b_ref[...], 0.0)
        cell = jnp.concatenate([xh, xnn], axis=1)
        cell_ref[...] = cell
        logits_ref[...] = jnp.dot(cell.astype(BF16), cw_ref[...].astype(BF16),
                                  preferred_element_type=F32) + cb_ref[...]


def _stage1(a1, a2, xs, xtf, xgene, ew, o, rg, aux, bsc, wblk):
    s = a1.shape[0]
    b = xs.shape[1]
    n = s // TILE
    zspec = pl.BlockSpec((TILE, b * D2), lambda i: (i, 0))
    return pl.pallas_call(
        _stage1_body,
        grid=(n,),
        in_specs=[
            pl.BlockSpec((TILE, a1.shape[1]), lambda i: (i, 0)),
            pl.BlockSpec((TILE, a2.shape[1]), lambda i: (i, 0)),
            pl.BlockSpec(xtf.shape, lambda i: (0, 0)),
            pl.BlockSpec(xgene.shape, lambda i: (0, 0)),
            pl.BlockSpec((TILE, b), lambda i: (i, 0)),
            pl.BlockSpec(ew.shape, lambda i: (0, 0)),
            pl.BlockSpec(o.shape, lambda i: (0, 0)),
            pl.BlockSpec(rg.shape, lambda i: (0, 0)),
            pl.BlockSpec(aux.shape, lambda i: (0, 0)),
            pl.BlockSpec(bsc.shape, lambda i: (0, 0)),
            pl.BlockSpec(wblk.shape, lambda i: (0, 0)),
        ],
        out_specs=[zspec, zspec, zspec,
                   pl.BlockSpec((TILE, a1.shape[1]), lambda i: (i, 0)),
                   pl.BlockSpec((TILE, a2.shape[1]), lambda i: (i, 0))],
        out_shape=[jax.ShapeDtypeStruct((s, b * D2), BF16),
                   jax.ShapeDtypeStruct((s, b * D2), BF16),
                   jax.ShapeDtypeStruct((s, b * D2), BF16),
                   jax.ShapeDtypeStruct((s, a1.shape[1]), BF16),
                   jax.ShapeDtypeStruct((s, a2.shape[1]), BF16)],
        compiler_params=pltpu.CompilerParams(
            dimension_semantics=("parallel",)),
    )(a1, a2, xtf, xgene, xs, ew, o, rg, aux, bsc, wblk)


def _stage2(a1, a2, ztf, zgene, selfc, o, rg, aux, b):
    s = a1.shape[0]
    n = s // TILE
    return pl.pallas_call(
        _stage2_body,
        grid=(n,),
        in_specs=[
            pl.BlockSpec((TILE, a1.shape[1]), lambda i: (i, 0)),
            pl.BlockSpec((TILE, a2.shape[1]), lambda i: (i, 0)),
            pl.BlockSpec(ztf.shape, lambda i: (0, 0)),
            pl.BlockSpec(zgene.shape, lambda i: (0, 0)),
            pl.BlockSpec((TILE, b * D2), lambda i: (i, 0)),
            pl.BlockSpec(o.shape, lambda i: (0, 0)),
            pl.BlockSpec(rg.shape, lambda i: (0, 0)),
            pl.BlockSpec(aux.shape, lambda i: (0, 0)),
        ],
        out_specs=pl.BlockSpec((TILE // 8, b * D2), lambda i: (i, 0)),
        out_shape=jax.ShapeDtypeStruct((s // 8, b * D2), BF16),
        compiler_params=pltpu.CompilerParams(
            dimension_semantics=("parallel",)),
    )(a1, a2, ztf, zgene, selfc, o, rg, aux)


def kernel(ft_tf, ft_gene, adj_tf_tf, adj_tf_gene, adj_gene_tf, adj_gene_gene, params):
    b = ft_tf.shape[0]
    xtf = ft_tf[:, :, 0].T
    xgene = ft_gene[:, :, 0].T
    xc = jnp.concatenate([ft_tf[:, :, 0], ft_gene[:, :, 0]], axis=1)

    p1 = params['hgc1']
    p2 = params['hgc2']
    eye = jnp.eye(b, dtype=F32)

    def pack_ew(k):
        w = jnp.stack([p1[k]['w_self'][0], p1[k]['w_rel_tf'][0],
                       p1[k]['w_rel_gene'][0]], axis=0) / 3.0
        e = jnp.einsum('jk,rd->rjkd', eye, w).reshape(3 * b, b * D1)
        bias_t = jnp.tile(p1[k]['bias'][0], b)[None]
        return jnp.concatenate([e, bias_t], axis=0)

    def pack_wblk(src):
        wc = jnp.concatenate(
            [p2['tf']['w_rel_' + src], p2['gene']['w_rel_' + src], p2[src]['w_self']],
            axis=1) / 3.0
        eye2 = jnp.eye(2, dtype=F32)
        ks = [jnp.einsum('jk,de->jdke', eye2, wc[:, c * D2:(c + 1) * D2])
              .reshape(2 * D1, 2 * D2) for c in range(3)]
        return jnp.concatenate(ks, axis=1).astype(BF16)

    o1 = jnp.repeat(eye, D1, axis=1).T / D1
    o2 = jnp.repeat(eye, D2, axis=1).T / D2
    rg1 = (jnp.repeat(eye, D1, axis=1) * jnp.tile(params['ln1_g'], b)[None]).astype(BF16)
    rg2 = (jnp.repeat(eye, D2, axis=1) * jnp.tile(params['ln2_g'], b)[None]).astype(BF16)

    aux1 = jnp.concatenate([jnp.tile(params['ln1_b'], b)[None],
                            jnp.zeros((7, b * D1), F32)], axis=0)
    aux2 = jnp.concatenate([jnp.tile(params['ln2_b'], b)[None],
                            jnp.zeros((7, b * D2), F32)], axis=0)

    def pack_bsc(k):
        return jnp.concatenate([jnp.tile(p2[k]['bias'][0], b)[None],
                                jnp.zeros((7, b * D2), F32)], axis=0)

    ztf_a, zgene_a, self_tf, a_tt, a_tg = _stage1(
        adj_tf_tf, adj_tf_gene, xtf, xtf, xgene,
        pack_ew('tf'), o1, rg1, aux1, pack_bsc('tf'), pack_wblk('tf'))
    ztf_b, zgene_b, self_gene, a_gt, a_gg = _stage1(
        adj_gene_tf, adj_gene_gene, xgene, xtf, xgene,
        pack_ew('gene'), o1, rg1, aux1, pack_bsc('gene'), pack_wblk('gene'))

    p_tf = _stage2(a_tt, a_tg, ztf_a, ztf_b, self_tf, o2, rg2, aux2, b)
    p_gene = _stage2(a_gt, a_gg, zgene_a, zgene_b, self_gene, o2, rg2, aux2, b)

    flat = (jnp.concatenate([p_tf, p_gene], axis=0)
            .reshape(-1, b, D2).transpose(1, 0, 2).reshape(b, -1))

    nk = 8
    kc = flat.shape[1] // nk
    logits, dec, cell = pl.pallas_call(
        _head_body,
        grid=(nk,),
        in_specs=[
            pl.BlockSpec((b, kc), lambda k: (0, k)),
            pl.BlockSpec(xc.shape, lambda k: (0, 0)),
            pl.BlockSpec((kc, D1), lambda k: (k, 0)),
            pl.BlockSpec((1, D1), lambda k: (0, 0)),
            pl.BlockSpec(params['rec_W'].shape, lambda k: (0, 0)),
            pl.BlockSpec((1, params['rec_b'].shape[0]), lambda k: (0, 0)),
            pl.BlockSpec(params['fc1_W'].shape, lambda k: (0, 0)),
            pl.BlockSpec((1, params['fc1_b'].shape[0]), lambda k: (0, 0)),
            pl.BlockSpec(params['fc2_W'].shape, lambda k: (0, 0)),
            pl.BlockSpec((1, params['fc2_b'].shape[0]), lambda k: (0, 0)),
            pl.BlockSpec(params['cls_W'].shape, lambda k: (0, 0)),
            pl.BlockSpec((1, params['cls_b'].shape[0]), lambda k: (0, 0)),
        ],
        out_specs=[
            pl.BlockSpec((b, params['cls_b'].shape[0]), lambda k: (0, 0)),
            pl.BlockSpec((b, params['rec_b'].shape[0]), lambda k: (0, 0)),
            pl.BlockSpec((b, 2 * params['fc2_b'].shape[0]), lambda k: (0, 0)),
        ],
        out_shape=[
            jax.ShapeDtypeStruct((b, params['cls_b'].shape[0]), F32),
            jax.ShapeDtypeStruct((b, params['rec_b'].shape[0]), F32),
            jax.ShapeDtypeStruct((b, 2 * params['fc2_b'].shape[0]), F32),
        ],
        scratch_shapes=[pltpu.VMEM((b, D1), F32)],
    )(flat, xc, params['ge_W'], params['ge_b'][None], params['rec_W'],
      params['rec_b'][None], params['fc1_W'], params['fc1_b'][None],
      params['fc2_W'], params['fc2_b'][None], params['cls_W'], params['cls_b'][None])
    return (logits, dec, cell)


# speedup vs baseline: 2.0931x; 1.0038x over previous
import jax
import jax.numpy as jnp
from jax.experimental import pallas as pl
from jax.experimental.pallas import tpu as pltpu

F32 = jnp.float32
BF16 = jnp.bfloat16
TILE = 384
TILE2 = 768
D1 = 128
D2 = 64


def _norm_elu(h, o, rg, lnb):
    s1 = jnp.dot(h, o, preferred_element_type=F32)
    s2 = jnp.dot(h * h, o, preferred_element_type=F32)
    rs = jax.lax.rsqrt(s2 - s1 * s1 + 1e-5)
    ab = jnp.dot(rs.astype(BF16), rg, preferred_element_type=F32)
    bb = jnp.dot((-s1 * rs).astype(BF16), rg, preferred_element_type=F32)
    xn = h * ab + bb + lnb
    return jnp.where(xn > 0, xn, jnp.exp(xn) - 1.0)


def _stage1_body(a1_ref, a2_ref, xtf_ref, xgene_ref, xs_ref, ew_ref, o_ref,
                 rg_ref, aux_ref, bsc_ref, wblk_ref,
                 ztf_ref, zgene_ref, selfc_ref, a1b_ref, a2b_ref):
    t = a1_ref.shape[0]
    a1b = a1_ref[...].astype(BF16)
    a2b = a2_ref[...].astype(BF16)
    a1b_ref[...] = a1b
    a2b_ref[...] = a2b
    ytf = jnp.dot(a1b, xtf_ref[...].astype(BF16), preferred_element_type=F32)
    ygene = jnp.dot(a2b, xgene_ref[...].astype(BF16), preferred_element_type=F32)
    ones = jnp.ones((t, 1), F32)
    c = jnp.concatenate([xs_ref[...], ytf, ygene, ones], axis=1)
    h = jnp.dot(c.astype(BF16), ew_ref[...].astype(BF16),
                preferred_element_type=F32)
    xn = _norm_elu(h, o_ref[...], rg_ref[...], aux_ref[0:1, :])
    xnb = xn.astype(BF16)
    wp = wblk_ref[...]
    b = xn.shape[1] // D1
    for g in range(b // 2):
        zg = jnp.dot(xnb[:, g * 2 * D1:(g + 1) * 2 * D1], wp,
                     preferred_element_type=F32)
        ztf_ref[:, g * D1:(g + 1) * D1] = zg[:, 0:D1].astype(BF16)
        zgene_ref[:, g * D1:(g + 1) * D1] = zg[:, D1:2 * D1].astype(BF16)
        selfc_ref[:, g * D1:(g + 1) * D1] = (
            zg[:, 2 * D1:3 * D1] + bsc_ref[0:1, g * D1:(g + 1) * D1]).astype(BF16)


def _stage2_body(a1_ref, a2_ref, ztf_ref, zgene_ref, selfc_ref, o_ref, rg_ref,
                 aux_ref, p_ref):
    t = a1_ref.shape[0]
    msg = jnp.dot(a1_ref[...], ztf_ref[...], preferred_element_type=F32)
    msg = msg + jnp.dot(a2_ref[...], zgene_ref[...], preferred_element_type=F32)
    h = msg + selfc_ref[...].astype(F32)
    xn = _norm_elu(h, o_ref[...], rg_ref[...], aux_ref[0:1, :])
    p_ref[...] = jnp.max(xn.reshape(t // 8, 8, xn.shape[1]), axis=1).astype(BF16)


def _head_body(flat_ref, xc_ref, gew_ref, geb_ref, recw_ref, recb_ref,
               f1w_ref, f1b_ref, f2w_ref, f2b_ref, cw_ref, cb_ref,
               logits_ref, dec_ref, cell_ref, acc_ref):
    k = pl.program_id(0)
    nk = pl.num_programs(0)
    part = jnp.dot(flat_ref[...], gew_ref[...].astype(BF16),
                   preferred_element_type=F32)

    @pl.when(k == 0)
    def _():
        acc_ref[...] = part

    @pl.when(k > 0)
    def _():
        acc_ref[...] = acc_ref[...] + part

    @pl.when(k == nk - 1)
    def _():
        xh = jnp.maximum(acc_ref[...] + geb_ref[...], 0.0)
        dec_ref[...] = jnp.dot(xh.astype(BF16), recw_ref[...].astype(BF16),
                               preferred_element_type=F32) + recb_ref[...]
        xnn = jnp.dot(xc_ref[...].astype(BF16), f1w_ref[...].astype(BF16),
                      preferred_element_type=F32)
        xnn = jnp.maximum(xnn + f1b_ref[...], 0.0)
        xnn = jnp.dot(xnn.astype(BF16), f2w_ref[...].astype(BF16),
                      preferred_element_type=F32)
        xnn = jnp.maximum(xnn + f2b_ref[...], 0.0)
        cell = jnp.concatenate([xh, xnn], axis=1)
        cell_ref[...] = cell
        logits_ref[...] = jnp.dot(cell.astype(BF16), cw_ref[...].astype(BF16),
                                  preferred_element_type=F32) + cb_ref[...]


def _stage1(a1, a2, xs, xtf, xgene, ew, o, rg, aux, bsc, wblk):
    s = a1.shape[0]
    b = xs.shape[1]
    n = s // TILE
    zspec = pl.BlockSpec((TILE, b * D2), lambda i: (i, 0))
    return pl.pallas_call(
        _stage1_body,
        grid=(n,),
        in_specs=[
            pl.BlockSpec((TILE, a1.shape[1]), lambda i: (i, 0)),
            pl.BlockSpec((TILE, a2.shape[1]), lambda i: (i, 0)),
            pl.BlockSpec(xtf.shape, lambda i: (0, 0)),
            pl.BlockSpec(xgene.shape, lambda i: (0, 0)),
            pl.BlockSpec((TILE, b), lambda i: (i, 0)),
            pl.BlockSpec(ew.shape, lambda i: (0, 0)),
            pl.BlockSpec(o.shape, lambda i: (0, 0)),
            pl.BlockSpec(rg.shape, lambda i: (0, 0)),
            pl.BlockSpec(aux.shape, lambda i: (0, 0)),
            pl.BlockSpec(bsc.shape, lambda i: (0, 0)),
            pl.BlockSpec(wblk.shape, lambda i: (0, 0)),
        ],
        out_specs=[zspec, zspec, zspec,
                   pl.BlockSpec((TILE, a1.shape[1]), lambda i: (i, 0)),
                   pl.BlockSpec((TILE, a2.shape[1]), lambda i: (i, 0))],
        out_shape=[jax.ShapeDtypeStruct((s, b * D2), BF16),
                   jax.ShapeDtypeStruct((s, b * D2), BF16),
                   jax.ShapeDtypeStruct((s, b * D2), BF16),
                   jax.ShapeDtypeStruct((s, a1.shape[1]), BF16),
                   jax.ShapeDtypeStruct((s, a2.shape[1]), BF16)],
        compiler_params=pltpu.CompilerParams(
            dimension_semantics=("parallel",)),
    )(a1, a2, xtf, xgene, xs, ew, o, rg, aux, bsc, wblk)


def _stage2(a1, a2, ztf, zgene, selfc, o, rg, aux, b):
    s = a1.shape[0]
    tile = min(TILE2, s)
    n = s // tile
    return pl.pallas_call(
        _stage2_body,
        grid=(n,),
        in_specs=[
            pl.BlockSpec((tile, a1.shape[1]), lambda i: (i, 0)),
            pl.BlockSpec((tile, a2.shape[1]), lambda i: (i, 0)),
            pl.BlockSpec(ztf.shape, lambda i: (0, 0)),
            pl.BlockSpec(zgene.shape, lambda i: (0, 0)),
            pl.BlockSpec((tile, b * D2), lambda i: (i, 0)),
            pl.BlockSpec(o.shape, lambda i: (0, 0)),
            pl.BlockSpec(rg.shape, lambda i: (0, 0)),
            pl.BlockSpec(aux.shape, lambda i: (0, 0)),
        ],
        out_specs=pl.BlockSpec((tile // 8, b * D2), lambda i: (i, 0)),
        out_shape=jax.ShapeDtypeStruct((s // 8, b * D2), BF16),
        compiler_params=pltpu.CompilerParams(
            dimension_semantics=("parallel",)),
    )(a1, a2, ztf, zgene, selfc, o, rg, aux)


def kernel(ft_tf, ft_gene, adj_tf_tf, adj_tf_gene, adj_gene_tf, adj_gene_gene, params):
    b = ft_tf.shape[0]
    xtf = ft_tf[:, :, 0].T
    xgene = ft_gene[:, :, 0].T
    xc = jnp.concatenate([ft_tf[:, :, 0], ft_gene[:, :, 0]], axis=1)

    p1 = params['hgc1']
    p2 = params['hgc2']
    eye = jnp.eye(b, dtype=F32)

    def pack_ew(k):
        w = jnp.stack([p1[k]['w_self'][0], p1[k]['w_rel_tf'][0],
                       p1[k]['w_rel_gene'][0]], axis=0) / 3.0
        e = jnp.einsum('jk,rd->rjkd', eye, w).reshape(3 * b, b * D1)
        bias_t = jnp.tile(p1[k]['bias'][0], b)[None]
        return jnp.concatenate([e, bias_t], axis=0)

    def pack_wblk(src):
        wc = jnp.concatenate(
            [p2['tf']['w_rel_' + src], p2['gene']['w_rel_' + src], p2[src]['w_self']],
            axis=1) / 3.0
        eye2 = jnp.eye(2, dtype=F32)
        ks = [jnp.einsum('jk,de->jdke', eye2, wc[:, c * D2:(c + 1) * D2])
              .reshape(2 * D1, 2 * D2) for c in range(3)]
        return jnp.concatenate(ks, axis=1).astype(BF16)

    o1 = jnp.repeat(eye, D1, axis=1).T / D1
    o2 = jnp.repeat(eye, D2, axis=1).T / D2
    rg1 = (jnp.repeat(eye, D1, axis=1) * jnp.tile(params['ln1_g'], b)[None]).astype(BF16)
    rg2 = (jnp.repeat(eye, D2, axis=1) * jnp.tile(params['ln2_g'], b)[None]).astype(BF16)

    aux1 = jnp.concatenate([jnp.tile(params['ln1_b'], b)[None],
                            jnp.zeros((7, b * D1), F32)], axis=0)
    aux2 = jnp.concatenate([jnp.tile(params['ln2_b'], b)[None],
                            jnp.zeros((7, b * D2), F32)], axis=0)

    def pack_bsc(k):
        return jnp.concatenate([jnp.tile(p2[k]['bias'][0], b)[None],
                                jnp.zeros((7, b * D2), F32)], axis=0)

    ztf_a, zgene_a, self_tf, a_tt, a_tg = _stage1(
        adj_tf_tf, adj_tf_gene, xtf, xtf, xgene,
        pack_ew('tf'), o1, rg1, aux1, pack_bsc('tf'), pack_wblk('tf'))
    ztf_b, zgene_b, self_gene, a_gt, a_gg = _stage1(
        adj_gene_tf, adj_gene_gene, xgene, xtf, xgene,
        pack_ew('gene'), o1, rg1, aux1, pack_bsc('gene'), pack_wblk('gene'))

    p_tf = _stage2(a_tt, a_tg, ztf_a, ztf_b, self_tf, o2, rg2, aux2, b)
    p_gene = _stage2(a_gt, a_gg, zgene_a, zgene_b, self_gene, o2, rg2, aux2, b)

    flat = (jnp.concatenate([p_tf, p_gene], axis=0)
            .reshape(-1, b, D2).transpose(1, 0, 2).reshape(b, -1))

    nk = 8
    kc = flat.shape[1] // nk
    logits, dec, cell = pl.pallas_call(
        _head_body,
        grid=(nk,),
        in_specs=[
            pl.BlockSpec((b, kc), lambda k: (0, k)),
            pl.BlockSpec(xc.shape, lambda k: (0, 0)),
            pl.BlockSpec((kc, D1), lambda k: (k, 0)),
            pl.BlockSpec((1, D1), lambda k: (0, 0)),
            pl.BlockSpec(params['rec_W'].shape, lambda k: (0, 0)),
            pl.BlockSpec((1, params['rec_b'].shape[0]), lambda k: (0, 0)),
            pl.BlockSpec(params['fc1_W'].shape, lambda k: (0, 0)),
            pl.BlockSpec((1, params['fc1_b'].shape[0]), lambda k: (0, 0)),
            pl.BlockSpec(params['fc2_W'].shape, lambda k: (0, 0)),
            pl.BlockSpec((1, params['fc2_b'].shape[0]), lambda k: (0, 0)),
            pl.BlockSpec(params['cls_W'].shape, lambda k: (0, 0)),
            pl.BlockSpec((1, params['cls_b'].shape[0]), lambda k: (0, 0)),
        ],
        out_specs=[
            pl.BlockSpec((b, params['cls_b'].shape[0]), lambda k: (0, 0)),
            pl.BlockSpec((b, params['rec_b'].shape[0]), lambda k: (0, 0)),
            pl.BlockSpec((b, 2 * params['fc2_b'].shape[0]), lambda k: (0, 0)),
        ],
        out_shape=[
            jax.ShapeDtypeStruct((b, params['cls_b'].shape[0]), F32),
            jax.ShapeDtypeStruct((b, params['rec_b'].shape[0]), F32),
            jax.ShapeDtypeStruct((b, 2 * params['fc2_b'].shape[0]), F32),
        ],
        scratch_shapes=[pltpu.VMEM((b, D1), F32)],
    )(flat, xc, params['ge_W'], params['ge_b'][None], params['rec_W'],
      params['rec_b'][None], params['fc1_W'], params['fc1_b'][None],
      params['fc2_W'], params['fc2_b'][None], params['cls_W'], params['cls_b'][None])
    return (logits, dec, cell)


# speedup vs baseline: 2.1501x; 1.0273x over previous
import jax
import jax.numpy as jnp
from jax.experimental import pallas as pl
from jax.experimental.pallas import tpu as pltpu

F32 = jnp.float32
BF16 = jnp.bfloat16
TILE = 384
D1 = 128
D2 = 64


def _norm_elu(h, o, rg, lnb):
    s1 = jnp.dot(h, o, preferred_element_type=F32)
    s2 = jnp.dot(h * h, o, preferred_element_type=F32)
    rs = jax.lax.rsqrt(s2 - s1 * s1 + 1e-5)
    ab = jnp.dot(rs.astype(BF16), rg, preferred_element_type=F32)
    bb = jnp.dot((-s1 * rs).astype(BF16), rg, preferred_element_type=F32)
    xn = h * ab + bb + lnb
    return jnp.where(xn > 0, xn, jnp.exp(xn) - 1.0)


def _stage1_body(att_ref, atg_ref, agt_ref, agg_ref, xbd_ref, xs_ref, ew_ref,
                 o_ref, rg_ref, aux_ref, bsc_ref, wblk_ref,
                 z_ref, selfc_ref, ab_ref):
    i = pl.program_id(0)
    t = xs_ref.shape[0]
    ntf = att_ref.shape[1]

    @pl.when(i == 0)
    def _():
        ab_ref[:, 0:ntf] = att_ref[...].astype(BF16)
        ab_ref[:, ntf:] = atg_ref[...].astype(BF16)

    @pl.when(i > 0)
    def _():
        ab_ref[:, 0:ntf] = agt_ref[...].astype(BF16)
        ab_ref[:, ntf:] = agg_ref[...].astype(BF16)

    y2 = jnp.dot(ab_ref[...], xbd_ref[...], preferred_element_type=F32)
    ones = jnp.ones((t, 1), F32)
    c = jnp.concatenate([xs_ref[...], y2, ones], axis=1)
    h = jnp.dot(c.astype(BF16), ew_ref[0], preferred_element_type=F32)
    xn = _norm_elu(h, o_ref[...], rg_ref[...], aux_ref[0:1, :])
    xnb = xn.astype(BF16)
    wp = wblk_ref[0]
    b = xn.shape[1] // D1
    q = b * D2
    for g in range(b // 2):
        zg = jnp.dot(xnb[:, g * 2 * D1:(g + 1) * 2 * D1], wp,
                     preferred_element_type=F32)
        z_ref[:, g * D1:(g + 1) * D1] = zg[:, 0:D1].astype(BF16)
        z_ref[:, q + g * D1:q + (g + 1) * D1] = zg[:, D1:2 * D1].astype(BF16)
        selfc_ref[:, g * D1:(g + 1) * D1] = (
            zg[:, 2 * D1:3 * D1] + bsc_ref[0, 0:1, g * D1:(g + 1) * D1]).astype(BF16)


def _stage2_body(ab_ref, z_ref, selfc_ref, o_ref, rg_ref, aux_ref, p_ref):
    t = ab_ref.shape[0]
    msg = jnp.dot(ab_ref[...], z_ref[...], preferred_element_type=F32)
    h = msg + selfc_ref[...].astype(F32)
    xn = _norm_elu(h, o_ref[...], rg_ref[...], aux_ref[0:1, :])
    p_ref[...] = jnp.max(xn.reshape(t // 8, 8, xn.shape[1]), axis=1).astype(BF16)


def _head_body(flat_ref, xc_ref, gew_ref, geb_ref, recw_ref, recb_ref,
               f1w_ref, f1b_ref, f2w_ref, f2b_ref, cw_ref, cb_ref,
               logits_ref, dec_ref, cell_ref, acc_ref):
    k = pl.program_id(0)
    nk = pl.num_programs(0)
    part = jnp.dot(flat_ref[...], gew_ref[...].astype(BF16),
                   preferred_element_type=F32)

    @pl.when(k == 0)
    def _():
        acc_ref[...] = part

    @pl.when(k > 0)
    def _():
        acc_ref[...] = acc_ref[...] + part

    @pl.when(k == nk - 1)
    def _():
        xh = jnp.maximum(acc_ref[...] + geb_ref[...], 0.0)
        dec_ref[...] = jnp.dot(xh.astype(BF16), recw_ref[...].astype(BF16),
                               preferred_element_type=F32) + recb_ref[...]
        xnn = jnp.dot(xc_ref[...].astype(BF16), f1w_ref[...].astype(BF16),
                      preferred_element_type=F32)
        xnn = jnp.maximum(xnn + f1b_ref[...], 0.0)
        xnn = jnp.dot(xnn.astype(BF16), f2w_ref[...].astype(BF16),
                      preferred_element_type=F32)
        xnn = jnp.maximum(xnn + f2b_ref[...], 0.0)
        cell = jnp.concatenate([xh, xnn], axis=1)
        cell_ref[...] = cell
        logits_ref[...] = jnp.dot(cell.astype(BF16), cw_ref[...].astype(BF16),
                                  preferred_element_type=F32) + cb_ref[...]


def _tsel(i):
    return jnp.where(i == 0, 0, 1)


def kernel(ft_tf, ft_gene, adj_tf_tf, adj_tf_gene, adj_gene_tf, adj_gene_gene, params):
    b = ft_tf.shape[0]
    ntf = adj_tf_tf.shape[0]
    ngene = adj_gene_gene.shape[0]
    s = ntf + ngene
    n = s // TILE
    xtf = ft_tf[:, :, 0].T
    xgene = ft_gene[:, :, 0].T
    xall = jnp.concatenate([xtf, xgene], axis=0)
    xc = jnp.concatenate([ft_tf[:, :, 0], ft_gene[:, :, 0]], axis=1)

    p1 = params['hgc1']
    p2 = params['hgc2']
    eye = jnp.eye(b, dtype=F32)

    def pack_ew(k):
        w = jnp.stack([p1[k]['w_self'][0], p1[k]['w_rel_tf'][0],
                       p1[k]['w_rel_gene'][0]], axis=0) / 3.0
        e = jnp.einsum('jk,rd->rjkd', eye, w).reshape(3 * b, b * D1)
        bias_t = jnp.tile(p1[k]['bias'][0], b)[None]
        return jnp.concatenate([e, bias_t], axis=0)

    def pack_wblk(src):
        wc = jnp.concatenate(
            [p2['tf']['w_rel_' + src], p2['gene']['w_rel_' + src], p2[src]['w_self']],
            axis=1) / 3.0
        eye2 = jnp.eye(2, dtype=F32)
        ks = [jnp.einsum('jk,de->jdke', eye2, wc[:, c * D2:(c + 1) * D2])
              .reshape(2 * D1, 2 * D2) for c in range(3)]
        return jnp.concatenate(ks, axis=1).astype(BF16)

    def pack_bsc(k):
        return jnp.concatenate([jnp.tile(p2[k]['bias'][0], b)[None],
                                jnp.zeros((7, b * D2), F32)], axis=0)

    ew = jnp.stack([pack_ew('tf'), pack_ew('gene')]).astype(BF16)
    wblk = jnp.stack([pack_wblk('tf'), pack_wblk('gene')])
    bsc = jnp.stack([pack_bsc('tf'), pack_bsc('gene')])

    o1 = jnp.repeat(eye, D1, axis=1).T / D1
    o2 = jnp.repeat(eye, D2, axis=1).T / D2
    rg1 = (jnp.repeat(eye, D1, axis=1) * jnp.tile(params['ln1_g'], b)[None]).astype(BF16)
    rg2 = (jnp.repeat(eye, D2, axis=1) * jnp.tile(params['ln2_g'], b)[None]).astype(BF16)
    aux1 = jnp.concatenate([jnp.tile(params['ln1_b'], b)[None],
                            jnp.zeros((7, b * D1), F32)], axis=0)
    aux2 = jnp.concatenate([jnp.tile(params['ln2_b'], b)[None],
                            jnp.zeros((7, b * D2), F32)], axis=0)

    xbd = jnp.zeros((s, 2 * b), F32)
    xbd = xbd.at[0:ntf, 0:b].set(xtf).at[ntf:, b:].set(xgene).astype(BF16)

    z, selfc, a_bf = pl.pallas_call(
        _stage1_body,
        grid=(n,),
        in_specs=[
            pl.BlockSpec(adj_tf_tf.shape, lambda i: (0, 0)),
            pl.BlockSpec(adj_tf_gene.shape, lambda i: (0, 0)),
            pl.BlockSpec((TILE, ntf), lambda i: (jnp.maximum(i - 1, 0), 0)),
            pl.BlockSpec((TILE, ngene), lambda i: (jnp.maximum(i - 1, 0), 0)),
            pl.BlockSpec(xbd.shape, lambda i: (0, 0)),
            pl.BlockSpec((TILE, b), lambda i: (i, 0)),
            pl.BlockSpec((1,) + ew.shape[1:], lambda i: (_tsel(i), 0, 0)),
            pl.BlockSpec(o1.shape, lambda i: (0, 0)),
            pl.BlockSpec(rg1.shape, lambda i: (0, 0)),
            pl.BlockSpec(aux1.shape, lambda i: (0, 0)),
            pl.BlockSpec((1,) + bsc.shape[1:], lambda i: (_tsel(i), 0, 0)),
            pl.BlockSpec((1,) + wblk.shape[1:], lambda i: (_tsel(i), 0, 0)),
        ],
        out_specs=[
            pl.BlockSpec((TILE, 2 * b * D2), lambda i: (i, 0)),
            pl.BlockSpec((TILE, b * D2), lambda i: (i, 0)),
            pl.BlockSpec((TILE, s), lambda i: (i, 0)),
        ],
        out_shape=[jax.ShapeDtypeStruct((s, 2 * b * D2), BF16),
                   jax.ShapeDtypeStruct((s, b * D2), BF16),
                   jax.ShapeDtypeStruct((s, s), BF16)],
    )(adj_tf_tf, adj_tf_gene, adj_gene_tf, adj_gene_gene,
      xbd, xall, ew, o1, rg1, aux1, bsc, wblk)

    p = pl.pallas_call(
        _stage2_body,
        grid=(n,),
        in_specs=[
            pl.BlockSpec((TILE, s), lambda i: (i, 0)),
            pl.BlockSpec((s, b * D2), lambda i: (0, _tsel(i))),
            pl.BlockSpec((TILE, b * D2), lambda i: (i, 0)),
            pl.BlockSpec(o2.shape, lambda i: (0, 0)),
            pl.BlockSpec(rg2.shape, lambda i: (0, 0)),
            pl.BlockSpec(aux2.shape, lambda i: (0, 0)),
        ],
        out_specs=pl.BlockSpec((TILE // 8, b * D2), lambda i: (i, 0)),
        out_shape=jax.ShapeDtypeStruct((s // 8, b * D2), BF16),
    )(a_bf, z, selfc, o2, rg2, aux2)

    flat = p.reshape(-1, b, D2).transpose(1, 0, 2).reshape(b, -1)

    nk = 8
    kc = flat.shape[1] // nk
    logits, dec, cell = pl.pallas_call(
        _head_body,
        grid=(nk,),
        in_specs=[
            pl.BlockSpec((b, kc), lambda k: (0, k)),
            pl.BlockSpec(xc.shape, lambda k: (0, 0)),
            pl.BlockSpec((kc, D1), lambda k: (k, 0)),
            pl.BlockSpec((1, D1), lambda k: (0, 0)),
            pl.BlockSpec(params['rec_W'].shape, lambda k: (0, 0)),
            pl.BlockSpec((1, params['rec_b'].shape[0]), lambda k: (0, 0)),
            pl.BlockSpec(params['fc1_W'].shape, lambda k: (0, 0)),
            pl.BlockSpec((1, params['fc1_b'].shape[0]), lambda k: (0, 0)),
            pl.BlockSpec(params['fc2_W'].shape, lambda k: (0, 0)),
            pl.BlockSpec((1, params['fc2_b'].shape[0]), lambda k: (0, 0)),
            pl.BlockSpec(params['cls_W'].shape, lambda k: (0, 0)),
            pl.BlockSpec((1, params['cls_b'].shape[0]), lambda k: (0, 0)),
        ],
        out_specs=[
            pl.BlockSpec((b, params['cls_b'].shape[0]), lambda k: (0, 0)),
            pl.BlockSpec((b, params['rec_b'].shape[0]), lambda k: (0, 0)),
            pl.BlockSpec((b, 2 * params['fc2_b'].shape[0]), lambda k: (0, 0)),
        ],
        out_shape=[
            jax.ShapeDtypeStruct((b, params['cls_b'].shape[0]), F32),
            jax.ShapeDtypeStruct((b, params['rec_b'].shape[0]), F32),
            jax.ShapeDtypeStruct((b, 2 * params['fc2_b'].shape[0]), F32),
        ],
        scratch_shapes=[pltpu.VMEM((b, D1), F32)],
    )(flat, xc, params['ge_W'], params['ge_b'][None], params['rec_W'],
      params['rec_b'][None], params['fc1_W'], params['fc1_b'][None],
      params['fc2_W'], params['fc2_b'][None], params['cls_W'], params['cls_b'][None])
    return (logits, dec, cell)
